# baseline (speedup 1.0000x reference)
import functools
import jax
import jax.numpy as jnp
from jax import lax
from jax.experimental import pallas as pl
from jax.experimental.pallas import tpu as pltpu
from jax.experimental.pallas import tpu_sc as plsc

_D = 128
_N = 100
_B = 1024
_BB = 32
_TEMP = 1.0

_NC = 2
_NS = 16
_NW = _NC * _NS
_RPW = _B // _NW
_DD = _D * _D
_C = 4


def _table_body(w_ref, deltas_ref, twb_ref):
    w = w_ref[...]
    twb_ref[0:_N] = w[None] + deltas_ref[...]
    twb_ref[_N] = w


def _build_table(W_adj, env_deltas):
    return pl.pallas_call(
        _table_body,
        out_shape=jax.ShapeDtypeStruct((_N + 1, _D, _D), jnp.float32),
    )(W_adj, env_deltas)


def _a_body(env_idx_ref, twb_ref, a_ref, ta_ref):
    i = pl.program_id(0)

    @pl.when(i == 0)
    def _tables():
        row = jax.lax.broadcasted_iota(jnp.int32, (_D, _D), 0)
        col = jax.lax.broadcasted_iota(jnp.int32, (_D, _D), 1)
        mask = jnp.where(row == col, 0.0, 1.0)
        ta_ref[...] = jax.nn.sigmoid(twb_ref[...] * (1.0 / _TEMP)) * mask[None]

    @pl.when(i > 0)
    def _emit():
        base = (i - 1) * _BB
        for j in range(_BB):
            e = env_idx_ref[base + j]
            a_ref[j] = ta_ref[jnp.clip(e, 0, _N)]


def _a_stream(env_idx, twb):
    return pl.pallas_call(
        _a_body,
        grid=(1 + _B // _BB,),
        in_specs=[
            pl.BlockSpec(memory_space=pltpu.SMEM),
            pl.BlockSpec((_N + 1, _D, _D), lambda i: (0, 0, 0)),
        ],
        out_specs=pl.BlockSpec((_BB, _D, _D), lambda i: (jnp.maximum(i - 1, 0), 0, 0)),
        out_shape=jax.ShapeDtypeStruct((_B, _D, _D), jnp.float32),
        scratch_shapes=[pltpu.VMEM((_N + 1, _D, _D), jnp.float32)],
    )(env_idx, twb)


def _sc_body(idx_hbm, twb_hbm, wb_hbm, idx_v, rows_v, sem):
    c = lax.axis_index("c")
    s = lax.axis_index("s")
    wid = s * _NC + c
    base = wid * _RPW

    pltpu.sync_copy(idx_hbm.at[wid], idx_v)
    for ch in range(_RPW // _C):
        pltpu.async_copy(twb_hbm.at[idx_v.at[ch]], rows_v, sem).wait()
        pltpu.sync_copy(rows_v, wb_hbm.at[pl.ds(base + ch * _C, _C)])


@functools.partial(
    pl.kernel,
    out_type=jax.ShapeDtypeStruct((_B, _DD), jnp.float32),
    mesh=plsc.VectorSubcoreMesh(
        core_axis_name="c", subcore_axis_name="s", num_cores=_NC, num_subcores=_NS
    ),
    scratch_types=[
        pltpu.VMEM((_RPW // _C, _C), jnp.int32),
        pltpu.VMEM((_C, _DD), jnp.float32),
        pltpu.SemaphoreType.DMA,
    ],
)
def _wb_stream(idx_hbm, twb_hbm, wb_hbm, idx_v, rows_v, sem):
    _sc_body(idx_hbm, twb_hbm, wb_hbm, idx_v, rows_v, sem)


@jax.jit
def _run(env_idx, W_adj, env_deltas):
    eidx = jnp.clip(env_idx, 0, _N).astype(jnp.int32)
    twb = _build_table(W_adj, env_deltas)
    A = _a_stream(eidx, twb)
    Wb = _wb_stream(eidx.reshape(_NW, _RPW // _C, _C), twb.reshape(_N + 1, _DD))
    return A, Wb.reshape(_B, _D, _D)


def kernel(env_idx, W_adj, env_deltas):
    return _run(env_idx, W_adj, env_deltas)


# speedup vs baseline: 1.2170x; 1.2170x over previous
import functools
import jax
import jax.numpy as jnp
from jax import lax
from jax.experimental import pallas as pl
from jax.experimental.pallas import tpu as pltpu
from jax.experimental.pallas import tpu_sc as plsc

_D = 128
_N = 100
_B = 1024
_BB = 32
_TEMP = 1.0

_NC = 2
_NS = 16
_NW = _NC * _NS
_RPW = _B // _NW
_DD = _D * _D
_C = 4


def _table_body(w_ref, deltas_ref, twb_ref):
    w = w_ref[...]
    twb_ref[0:_N] = w[None] + deltas_ref[...]
    twb_ref[_N] = w


def _build_table(W_adj, env_deltas):
    return pl.pallas_call(
        _table_body,
        out_shape=jax.ShapeDtypeStruct((_N + 1, _D, _D), jnp.float32),
    )(W_adj, env_deltas)


def _a_body(env_idx_ref, twb_ref, a_ref, ta_ref):
    i = pl.program_id(0)

    @pl.when(i == 0)
    def _tables():
        row = jax.lax.broadcasted_iota(jnp.int32, (_D, _D), 0)
        col = jax.lax.broadcasted_iota(jnp.int32, (_D, _D), 1)
        mask = jnp.where(row == col, 0.0, 1.0)
        ta_ref[...] = jax.nn.sigmoid(twb_ref[...] * (1.0 / _TEMP)) * mask[None]

    @pl.when(i > 0)
    def _emit():
        base = (i - 1) * _BB
        for j in range(_BB):
            e = env_idx_ref[base + j]
            a_ref[j] = ta_ref[jnp.clip(e, 0, _N)]


def _a_stream(env_idx, twb):
    return pl.pallas_call(
        _a_body,
        grid=(1 + _B // _BB,),
        in_specs=[
            pl.BlockSpec(memory_space=pltpu.SMEM),
            pl.BlockSpec((_N + 1, _D, _D), lambda i: (0, 0, 0)),
        ],
        out_specs=pl.BlockSpec((_BB, _D, _D), lambda i: (jnp.maximum(i - 1, 0), 0, 0)),
        out_shape=jax.ShapeDtypeStruct((_B, _D, _D), jnp.float32),
        scratch_shapes=[pltpu.VMEM((_N + 1, _D, _D), jnp.float32)],
    )(env_idx, twb)


def _sc_body(idx_hbm, twb_hbm, wb_hbm, idx_v, rows_v, sem):
    c = lax.axis_index("c")
    s = lax.axis_index("s")
    wid = s * _NC + c
    base = wid * _RPW

    pltpu.sync_copy(idx_hbm.at[wid], idx_v)
    for ch in range(_RPW // _C):
        pltpu.async_copy(twb_hbm.at[idx_v.at[ch]], rows_v, sem).wait()
        pltpu.sync_copy(rows_v, wb_hbm.at[pl.ds(base + ch * _C, _C)])


@functools.partial(
    pl.kernel,
    out_type=jax.ShapeDtypeStruct((_B, _D, _D), jnp.float32),
    mesh=plsc.VectorSubcoreMesh(
        core_axis_name="c", subcore_axis_name="s", num_cores=_NC, num_subcores=_NS
    ),
    scratch_types=[
        pltpu.VMEM((_RPW // _C, _C), jnp.int32),
        pltpu.VMEM((_C, _D, _D), jnp.float32),
        pltpu.SemaphoreType.DMA,
    ],
)
def _wb_stream(idx_hbm, twb_hbm, wb_hbm, idx_v, rows_v, sem):
    _sc_body(idx_hbm, twb_hbm, wb_hbm, idx_v, rows_v, sem)


@jax.jit
def _run(env_idx, W_adj, env_deltas):
    eidx = jnp.clip(env_idx, 0, _N).astype(jnp.int32)
    twb = _build_table(W_adj, env_deltas)
    A = _a_stream(eidx, twb)
    Wb = _wb_stream(eidx.reshape(_NW, _RPW // _C, _C), twb)
    return A, Wb


def kernel(env_idx, W_adj, env_deltas):
    return _run(env_idx, W_adj, env_deltas)


# speedup vs baseline: 2.6383x; 2.1678x over previous
import jax
import jax.numpy as jnp
from jax.experimental import pallas as pl
from jax.experimental.pallas import tpu as pltpu

_D = 128
_N = 100
_B = 1024
_BB = 32
_TEMP = 1.0


def _body(env_idx_ref, w_ref, deltas_ref, a_ref, wb_ref, ta_ref):
    i = pl.program_id(0)

    @pl.when(i == 0)
    def _tables():
        w = w_ref[...]
        row = jax.lax.broadcasted_iota(jnp.int32, (_D, _D), 0)
        col = jax.lax.broadcasted_iota(jnp.int32, (_D, _D), 1)
        mask = jnp.where(row == col, 0.0, 1.0)
        ta_ref[0:_N] = jax.nn.sigmoid((w[None] + deltas_ref[...]) * (1.0 / _TEMP)) * mask[None]
        ta_ref[_N] = jax.nn.sigmoid(w * (1.0 / _TEMP)) * mask

    @pl.when(i > 0)
    def _emit():
        w = w_ref[...]
        base = (i - 1) * _BB
        for j in range(_BB):
            e = env_idx_ref[base + j]
            idx = jnp.clip(e, 0, _N - 1)
            valid = e < _N
            wb_ref[j] = w + jnp.where(valid, 1.0, 0.0) * deltas_ref[idx]
            a_ref[j] = ta_ref[jnp.where(valid, idx, _N)]


@jax.jit
def _run(env_idx, W_adj, env_deltas):
    grid = (1 + _B // _BB,)
    out_shape = (
        jax.ShapeDtypeStruct((_B, _D, _D), jnp.float32),
        jax.ShapeDtypeStruct((_B, _D, _D), jnp.float32),
    )
    out_map = lambda i: (jnp.maximum(i - 1, 0), 0, 0)
    return pl.pallas_call(
        _body,
        grid=grid,
        in_specs=[
            pl.BlockSpec(memory_space=pltpu.SMEM),
            pl.BlockSpec((_D, _D), lambda i: (0, 0)),
            pl.BlockSpec((_N, _D, _D), lambda i: (0, 0, 0)),
        ],
        out_specs=[
            pl.BlockSpec((_BB, _D, _D), out_map),
            pl.BlockSpec((_BB, _D, _D), out_map),
        ],
        out_shape=out_shape,
        scratch_shapes=[pltpu.VMEM((_N + 1, _D, _D), jnp.float32)],
    )(env_idx, W_adj, env_deltas)


def kernel(env_idx, W_adj, env_deltas):
    return _run(env_idx, W_adj, env_deltas)
